# baseline (speedup 1.0000x reference)
import functools

import jax
import jax.numpy as jnp
import numpy as np
from jax.experimental import pallas as pl

_U = jnp.uint32
_ROT_A = (13, 15, 26, 6)
_ROT_B = (17, 29, 16, 24)


def _rotl(x, d):
    return (x << _U(d)) | (x >> _U(32 - d))


def _rounds(x0, x1, rots):
    for r in rots:
        x0 = x0 + x1
        x1 = _rotl(x1, r)
        x1 = x0 ^ x1
    return x0, x1


def _threefry_bits(flat_u32, k1: int, k2: int):
    ks0 = _U(k1)
    ks1 = _U(k2)
    ks2 = _U(k1 ^ k2 ^ 0x1BD11BDA)
    x0 = jnp.full_like(flat_u32, ks0)
    x1 = flat_u32 + ks1
    x0, x1 = _rounds(x0, x1, _ROT_A)
    x0 = x0 + ks1
    x1 = x1 + ks2 + _U(1)
    x0, x1 = _rounds(x0, x1, _ROT_B)
    x0 = x0 + ks2
    x1 = x1 + ks0 + _U(2)
    x0, x1 = _rounds(x0, x1, _ROT_A)
    x0 = x0 + ks0
    x1 = x1 + ks1 + _U(3)
    x0, x1 = _rounds(x0, x1, _ROT_B)
    x0 = x0 + ks1
    x1 = x1 + ks2 + _U(4)
    x0, x1 = _rounds(x0, x1, _ROT_A)
    x0 = x0 + ks2
    x1 = x1 + ks0 + _U(5)
    return x0 ^ x1


def _gumbel_from_bits(bits):
    tiny = jnp.float32(np.finfo(np.float32).tiny)
    fb = (bits >> _U(9)) | _U(0x3F800000)
    f = jax.lax.bitcast_convert_type(fb, jnp.float32) - jnp.float32(1.0)
    u = jnp.maximum(tiny, f * (jnp.float32(1.0) - tiny) + tiny)
    return -jnp.log(-jnp.log(u))


def _kernel_body(x_ref, o_ref, *, rows, cols, k1, k2):
    x = x_ref[...]
    m = jnp.max(x, axis=1, keepdims=True)
    e = jnp.exp(x - m)
    s = e / jnp.sum(e, axis=1, keepdims=True)
    la = jnp.log(jnp.clip(s, jnp.float32(1e-10), jnp.float32(1.0)))

    col = jax.lax.broadcasted_iota(jnp.int32, (rows, cols), 1)
    row = jax.lax.broadcasted_iota(jnp.int32, (rows, cols), 0)
    flat = (pl.program_id(0) * rows + row) * cols + col
    g = _gumbel_from_bits(_threefry_bits(flat.astype(jnp.uint32), k1, k2))

    v = g + la
    vm = jnp.max(v, axis=1, keepdims=True)
    big = jnp.int32(cols)
    idx = jnp.min(jnp.where(v == vm, col, big), axis=1, keepdims=True)
    o_ref[...] = (col == idx).astype(jnp.float32)


_K1, _K2 = 0, 42


def _gumbel_softmax_sample(logits2d, rows=8, interpret=False):
    n, c = logits2d.shape
    body = functools.partial(_kernel_body, rows=rows, cols=c, k1=_K1, k2=_K2)
    return pl.pallas_call(
        body,
        grid=(n // rows,),
        in_specs=[pl.BlockSpec((rows, c), lambda i: (i, 0))],
        out_specs=pl.BlockSpec((rows, c), lambda i: (i, 0)),
        out_shape=jax.ShapeDtypeStruct((n, c), jnp.float32),
        interpret=interpret,
    )(logits2d)


def kernel(logits):
    b, t, c = logits.shape
    out = _gumbel_softmax_sample(logits.reshape(b * t, c))
    return out.reshape(b, t, c)


# speedup vs baseline: 1.5766x; 1.5766x over previous
import functools

import jax
import jax.numpy as jnp
import numpy as np
from jax.experimental import pallas as pl

_U = jnp.uint32
_ROT_A = (13, 15, 26, 6)
_ROT_B = (17, 29, 16, 24)


def _rotl(x, d):
    return (x << _U(d)) | (x >> _U(32 - d))


def _rounds(x0, x1, rots):
    for r in rots:
        x0 = x0 + x1
        x1 = _rotl(x1, r)
        x1 = x0 ^ x1
    return x0, x1


def _threefry_bits(flat_u32, k1: int, k2: int):
    ks0 = _U(k1)
    ks1 = _U(k2)
    ks2 = _U(k1 ^ k2 ^ 0x1BD11BDA)
    x0 = jnp.full_like(flat_u32, ks0)
    x1 = flat_u32 + ks1
    x0, x1 = _rounds(x0, x1, _ROT_A)
    x0 = x0 + ks1
    x1 = x1 + ks2 + _U(1)
    x0, x1 = _rounds(x0, x1, _ROT_B)
    x0 = x0 + ks2
    x1 = x1 + ks0 + _U(2)
    x0, x1 = _rounds(x0, x1, _ROT_A)
    x0 = x0 + ks0
    x1 = x1 + ks1 + _U(3)
    x0, x1 = _rounds(x0, x1, _ROT_B)
    x0 = x0 + ks1
    x1 = x1 + ks2 + _U(4)
    x0, x1 = _rounds(x0, x1, _ROT_A)
    x0 = x0 + ks2
    x1 = x1 + ks0 + _U(5)
    return x0 ^ x1


def _gumbel_from_bits(bits):
    tiny = jnp.float32(np.finfo(np.float32).tiny)
    fb = (bits >> _U(9)) | _U(0x3F800000)
    f = jax.lax.bitcast_convert_type(fb, jnp.float32) - jnp.float32(1.0)
    u = jnp.maximum(tiny, f * (jnp.float32(1.0) - tiny) + tiny)
    return -jnp.log(-jnp.log(u))


def _kernel_body(x_ref, o_ref, *, rows, cols, k1, k2, chunk):
    x = x_ref[...]
    m = jnp.max(x, axis=1, keepdims=True)
    e = jnp.exp(x - m)
    s = e / jnp.sum(e, axis=1, keepdims=True)
    la = jnp.log(jnp.clip(s, jnp.float32(1e-10), jnp.float32(1.0)))

    row = jax.lax.broadcasted_iota(jnp.int32, (rows, 1), 0)
    base = ((pl.program_id(0) * rows + row) * cols).astype(jnp.uint32)
    best_v = jnp.full((rows, 1), -jnp.inf, dtype=jnp.float32)
    best_i = jnp.full((rows, 1), cols, dtype=jnp.int32)
    for off in range(0, cols, chunk):
        w = min(chunk, cols - off)
        colc = jax.lax.broadcasted_iota(jnp.int32, (rows, w), 1)
        flat = base + colc.astype(jnp.uint32) + _U(off)
        g = _gumbel_from_bits(_threefry_bits(flat, k1, k2))
        v = g + la[:, off:off + w]
        vm = jnp.max(v, axis=1, keepdims=True)
        im = jnp.min(jnp.where(v == vm, colc, jnp.int32(cols)),
                     axis=1, keepdims=True) + jnp.int32(off)
        take = vm > best_v
        best_v = jnp.where(take, vm, best_v)
        best_i = jnp.where(take, im, best_i)

    col = jax.lax.broadcasted_iota(jnp.int32, (rows, cols), 1)
    o_ref[...] = (col == best_i).astype(jnp.float32)


_K1, _K2 = 0, 42


def _gumbel_softmax_sample(logits2d, rows=8, chunk=2048, interpret=False):
    n, c = logits2d.shape
    body = functools.partial(_kernel_body, rows=rows, cols=c, k1=_K1, k2=_K2,
                             chunk=chunk)
    return pl.pallas_call(
        body,
        grid=(n // rows,),
        in_specs=[pl.BlockSpec((rows, c), lambda i: (i, 0))],
        out_specs=pl.BlockSpec((rows, c), lambda i: (i, 0)),
        out_shape=jax.ShapeDtypeStruct((n, c), jnp.float32),
        interpret=interpret,
    )(logits2d)


def kernel(logits):
    b, t, c = logits.shape
    out = _gumbel_softmax_sample(logits.reshape(b * t, c))
    return out.reshape(b, t, c)
